# baseline (speedup 1.0000x reference)
import jax
import jax.numpy as jnp
from jax.experimental import pallas as pl
from jax.experimental.pallas import tpu as pltpu

B, T, D = 16, 2048, 1024
N_E, D_E = 8, 64
TBLK = 64
NT = T // TBLK
GP = 3 * 128
NR = B * N_E
CPB = TBLK // 16


def _phase_a_full(h_ref, mc_ref, c_ref, bih_ref, gxb_ref):
    x2 = h_ref[...].reshape(B * TBLK, D).astype(jnp.bfloat16)
    mm = jnp.dot(x2, mc_ref[...], preferred_element_type=jnp.float32)
    lg = mm[:, GP:GP + N_E]
    m = jnp.max(lg, axis=-1, keepdims=True)
    p = jnp.exp(lg - m)
    w2 = p / jnp.sum(p, axis=-1, keepdims=True)
    w3 = w2.reshape(B, TBLK, N_E)
    for g in range(3):
        pg = mm[:, 128 * g:128 * g + D_E] + c_ref[:, 128 * g:128 * g + D_E]
        pre3g = pg.reshape(B, TBLK, D_E)
        gx4 = (w3[..., None] * pre3g[:, :, None, :]
               + bih_ref[:, 128 * g:128 * g + D_E])
        gxb_ref[:, :, 128 * g:128 * g + D_E] = (
            jnp.transpose(gx4, (1, 0, 2, 3)).reshape(TBLK, NR, D_E))


def _phase_a_chunk(ci, h_ref, mc_ref, c_ref, bih_ref, gxb_ref):
    b = ci // CPB
    t0 = (ci % CPB) * 16
    xc = h_ref[b, pl.ds(t0, 16), :].astype(jnp.bfloat16)
    mm = jnp.dot(xc, mc_ref[...], preferred_element_type=jnp.float32)
    lg = mm[:, GP:GP + N_E]
    m = jnp.max(lg, axis=-1, keepdims=True)
    p = jnp.exp(lg - m)
    w = p / jnp.sum(p, axis=-1, keepdims=True)
    r0 = pl.multiple_of(b * N_E, N_E)
    for g in range(3):
        pg = mm[:, 128 * g:128 * g + D_E] + c_ref[:, 128 * g:128 * g + D_E]
        gx3 = (w[:, :, None] * pg[:, None, :]
               + bih_ref[:, 128 * g:128 * g + D_E])
        gxb_ref[pl.ds(t0, 16), pl.ds(r0, N_E), 128 * g:128 * g + D_E] = gx3


def _recurrence_step(t, st, gxb_ref, whh_ref, bhh_ref, out_ref):
    gx = gxb_ref[t]
    gh = jnp.dot(st.astype(jnp.bfloat16), whh_ref[...],
                 preferred_element_type=jnp.float32) + bhh_ref[...]
    s = gx + gh
    r = jax.nn.sigmoid(s[:, :D_E])
    z = jax.nn.sigmoid(s[:, 128:128 + D_E])
    n = jnp.tanh(gx[:, 256:256 + D_E] + r * gh[:, 256:256 + D_E])
    new = n + z * (st - n)
    out_ref[:, pl.ds(t, 1), :, :] = new.reshape(B, 1, N_E, D_E)
    return new


def _entity_kernel(h_ref, mc_ref, c_ref, bih_ref, bhh_ref, whh_ref, e0_ref,
                   out_ref, state_ref, gxa_ref, gxb2_ref):
    j = pl.program_id(0)

    @pl.when(j == 0)
    def _():
        _phase_a_full(h_ref, mc_ref, c_ref, bih_ref, gxa_ref)
        state_ref[...] = jnp.concatenate([e0_ref[...]] * B, axis=0)

    def _fused(read_ref, write_ref, do_prep):
        def body(t, st):
            new = _recurrence_step(t, st, read_ref, whh_ref, bhh_ref, out_ref)
            if do_prep:
                _phase_a_chunk(t, h_ref, mc_ref, c_ref, bih_ref, write_ref)
            return new
        st = jax.lax.fori_loop(0, TBLK, body, state_ref[...], unroll=4)
        state_ref[...] = st

    odd = jax.lax.rem(j, 2) == 1

    @pl.when(jnp.logical_and(j > 0, odd))
    def _():
        _fused(gxa_ref, gxb2_ref, True)

    @pl.when(jnp.logical_and(j > 0, jnp.logical_not(odd)))
    def _():
        _fused(gxb2_ref, gxa_ref, True)


def _pad_gates(a):
    z = jnp.zeros(a.shape[:-1] + (64,), a.dtype)
    return jnp.concatenate(
        [a[..., :64], z, a[..., 64:128], z, a[..., 128:192], z], axis=-1)


def kernel(h_seq, entity_keys, Wi, bi, W_ih, W_hh, b_ih, b_hh, e0):
    m_pre = _pad_gates((W_ih @ Wi).T)
    keys_t = entity_keys.T / jnp.sqrt(jnp.float32(D))
    mc = jnp.concatenate([m_pre, keys_t], axis=1).astype(jnp.bfloat16)
    c = _pad_gates((W_ih @ bi).reshape(1, 192))
    bih2 = _pad_gates(b_ih.reshape(1, 192))
    bhh2 = _pad_gates(b_hh.reshape(1, 192))
    whh_t = _pad_gates(W_hh.T).astype(jnp.bfloat16)

    stack = pl.pallas_call(
        _entity_kernel,
        grid=(NT + 1,),
        in_specs=[
            pl.BlockSpec((B, TBLK, D),
                         lambda j: (0, jnp.minimum(j, NT - 1), 0)),
            pl.BlockSpec((D, GP + N_E), lambda j: (0, 0)),
            pl.BlockSpec((1, GP), lambda j: (0, 0)),
            pl.BlockSpec((1, GP), lambda j: (0, 0)),
            pl.BlockSpec((1, GP), lambda j: (0, 0)),
            pl.BlockSpec((D_E, GP), lambda j: (0, 0)),
            pl.BlockSpec((N_E, D_E), lambda j: (0, 0)),
        ],
        out_specs=pl.BlockSpec((B, TBLK, N_E, D_E),
                               lambda j: (0, jnp.maximum(j - 1, 0), 0, 0)),
        out_shape=jax.ShapeDtypeStruct((B, T, N_E, D_E), jnp.float32),
        scratch_shapes=[
            pltpu.VMEM((NR, D_E), jnp.float32),
            pltpu.VMEM((TBLK, NR, GP), jnp.float32),
            pltpu.VMEM((TBLK, NR, GP), jnp.float32),
        ],
        compiler_params=pltpu.CompilerParams(
            dimension_semantics=("arbitrary",),
            vmem_limit_bytes=100 * 1024 * 1024,
        ),
    )(h_seq, mc, c, bih2, bhh2, whh_t, e0)

    entity_seq = stack.reshape(B, T, N_E * D_E)
    return entity_seq, stack


# speedup vs baseline: 1.1480x; 1.1480x over previous
import jax
import jax.numpy as jnp
from jax.experimental import pallas as pl
from jax.experimental.pallas import tpu as pltpu

B, T, D = 16, 2048, 1024
N_E, D_E = 8, 64
TBLK = 128
NT = T // TBLK
GP = 3 * 128
NR = B * N_E


def _entity_kernel(h_ref, mc_ref, c_ref, bih_ref, bhh_ref, whh_ref, e0_ref,
                   out_ref, state_ref, gxb_ref):
    j = pl.program_id(0)

    x2 = h_ref[...].reshape(B * TBLK, D).astype(jnp.bfloat16)
    mm = jnp.dot(x2, mc_ref[...], preferred_element_type=jnp.float32)
    lg = mm[:, GP:GP + N_E]
    m = jnp.max(lg, axis=-1, keepdims=True)
    p = jnp.exp(lg - m)
    w2 = p / jnp.sum(p, axis=-1, keepdims=True)
    w3 = w2.reshape(B, TBLK, N_E)

    for g in range(3):
        pg = mm[:, 128 * g:128 * g + D_E] + c_ref[:, 128 * g:128 * g + D_E]
        pre3g = pg.reshape(B, TBLK, D_E)
        gx4 = (w3[..., None] * pre3g[:, :, None, :]
               + bih_ref[:, 128 * g:128 * g + D_E])
        gxb_ref[:, :, 128 * g:128 * g + D_E] = (
            jnp.transpose(gx4, (1, 0, 2, 3)).reshape(TBLK, NR, D_E))

    @pl.when(j == 0)
    def _():
        state_ref[...] = jnp.concatenate([e0_ref[...]] * B, axis=0)

    def body(t, st):
        gx = gxb_ref[t]
        gh = jnp.dot(st.astype(jnp.bfloat16), whh_ref[...],
                     preferred_element_type=jnp.float32) + bhh_ref[...]
        s = gx + gh
        r = jax.nn.sigmoid(s[:, :D_E])
        z = jax.nn.sigmoid(s[:, 128:128 + D_E])
        n = jnp.tanh(gx[:, 256:256 + D_E] + r * gh[:, 256:256 + D_E])
        new = n + z * (st - n)
        out_ref[:, pl.ds(t, 1), :, :] = new.reshape(B, 1, N_E, D_E)
        return new

    st = jax.lax.fori_loop(0, TBLK, body, state_ref[...], unroll=8)
    state_ref[...] = st


def _pad_gates(a):
    z = jnp.zeros(a.shape[:-1] + (64,), a.dtype)
    return jnp.concatenate(
        [a[..., :64], z, a[..., 64:128], z, a[..., 128:192], z], axis=-1)


def kernel(h_seq, entity_keys, Wi, bi, W_ih, W_hh, b_ih, b_hh, e0):
    m_pre = _pad_gates((W_ih @ Wi).T)
    keys_t = entity_keys.T / jnp.sqrt(jnp.float32(D))
    mc = jnp.concatenate([m_pre, keys_t], axis=1).astype(jnp.bfloat16)
    c = _pad_gates((W_ih @ bi).reshape(1, 192))
    bih2 = _pad_gates(b_ih.reshape(1, 192))
    bhh2 = _pad_gates(b_hh.reshape(1, 192))
    whh_t = _pad_gates(W_hh.T).astype(jnp.bfloat16)

    stack = pl.pallas_call(
        _entity_kernel,
        grid=(NT,),
        in_specs=[
            pl.BlockSpec((B, TBLK, D), lambda j: (0, j, 0)),
            pl.BlockSpec((D, GP + N_E), lambda j: (0, 0)),
            pl.BlockSpec((1, GP), lambda j: (0, 0)),
            pl.BlockSpec((1, GP), lambda j: (0, 0)),
            pl.BlockSpec((1, GP), lambda j: (0, 0)),
            pl.BlockSpec((D_E, GP), lambda j: (0, 0)),
            pl.BlockSpec((N_E, D_E), lambda j: (0, 0)),
        ],
        out_specs=pl.BlockSpec((B, TBLK, N_E, D_E), lambda j: (0, j, 0, 0)),
        out_shape=jax.ShapeDtypeStruct((B, T, N_E, D_E), jnp.float32),
        scratch_shapes=[
            pltpu.VMEM((NR, D_E), jnp.float32),
            pltpu.VMEM((TBLK, NR, GP), jnp.float32),
        ],
        compiler_params=pltpu.CompilerParams(
            dimension_semantics=("arbitrary",),
            vmem_limit_bytes=100 * 1024 * 1024,
        ),
    )(h_seq, mc, c, bih2, bhh2, whh_t, e0)

    entity_seq = stack.reshape(B, T, N_E * D_E)
    return entity_seq, stack


# speedup vs baseline: 1.1539x; 1.0052x over previous
import jax
import jax.numpy as jnp
from jax.experimental import pallas as pl
from jax.experimental.pallas import tpu as pltpu

B, T, D = 16, 2048, 1024
N_E, D_E = 8, 64
TBLK = 64
NT = T // TBLK
GP = 3 * 128
NR = B * N_E


def _entity_kernel(h_ref, mc_ref, c_ref, bih_ref, bhh_ref, whh_ref, e0_ref,
                   out_ref, state_ref, gxb_ref):
    j = pl.program_id(0)

    x2 = h_ref[...].reshape(B * TBLK, D).astype(jnp.bfloat16)
    mm = jnp.dot(x2, mc_ref[...], preferred_element_type=jnp.float32)
    lg = mm[:, GP:GP + N_E]
    m = jnp.max(lg, axis=-1, keepdims=True)
    p = jnp.exp(lg - m)
    w2 = p / jnp.sum(p, axis=-1, keepdims=True)
    w3 = w2.reshape(B, TBLK, N_E)

    for g in range(3):
        pg = mm[:, 128 * g:128 * g + D_E] + c_ref[:, 128 * g:128 * g + D_E]
        pre3g = pg.reshape(B, TBLK, D_E)
        gx4 = (w3[..., None] * pre3g[:, :, None, :]
               + bih_ref[:, 128 * g:128 * g + D_E])
        gxb_ref[:, :, 128 * g:128 * g + D_E] = (
            jnp.transpose(gx4, (1, 0, 2, 3)).reshape(TBLK, NR, D_E))

    @pl.when(j == 0)
    def _():
        state_ref[...] = jnp.concatenate([e0_ref[...]] * B, axis=0)

    def body(t, st):
        gx = gxb_ref[t]
        gh = jnp.dot(st.astype(jnp.bfloat16), whh_ref[...],
                     preferred_element_type=jnp.float32) + bhh_ref[...]
        s = gx + gh
        r = jax.nn.sigmoid(s[:, :D_E])
        z = jax.nn.sigmoid(s[:, 128:128 + D_E])
        n = jnp.tanh(gx[:, 256:256 + D_E] + r * gh[:, 256:256 + D_E])
        new = n + z * (st - n)
        out_ref[:, pl.ds(t, 1), :, :] = new.reshape(B, 1, N_E, D_E)
        return new

    st = jax.lax.fori_loop(0, TBLK, body, state_ref[...], unroll=8)
    state_ref[...] = st


def _pad_gates(a):
    z = jnp.zeros(a.shape[:-1] + (64,), a.dtype)
    return jnp.concatenate(
        [a[..., :64], z, a[..., 64:128], z, a[..., 128:192], z], axis=-1)


def kernel(h_seq, entity_keys, Wi, bi, W_ih, W_hh, b_ih, b_hh, e0):
    m_pre = _pad_gates((W_ih @ Wi).T)
    keys_t = entity_keys.T / jnp.sqrt(jnp.float32(D))
    mc = jnp.concatenate([m_pre, keys_t], axis=1).astype(jnp.bfloat16)
    c = _pad_gates((W_ih @ bi).reshape(1, 192))
    bih2 = _pad_gates(b_ih.reshape(1, 192))
    bhh2 = _pad_gates(b_hh.reshape(1, 192))
    whh_t = _pad_gates(W_hh.T).astype(jnp.bfloat16)

    stack = pl.pallas_call(
        _entity_kernel,
        grid=(NT,),
        in_specs=[
            pl.BlockSpec((B, TBLK, D), lambda j: (0, j, 0)),
            pl.BlockSpec((D, GP + N_E), lambda j: (0, 0)),
            pl.BlockSpec((1, GP), lambda j: (0, 0)),
            pl.BlockSpec((1, GP), lambda j: (0, 0)),
            pl.BlockSpec((1, GP), lambda j: (0, 0)),
            pl.BlockSpec((D_E, GP), lambda j: (0, 0)),
            pl.BlockSpec((N_E, D_E), lambda j: (0, 0)),
        ],
        out_specs=pl.BlockSpec((B, TBLK, N_E, D_E), lambda j: (0, j, 0, 0)),
        out_shape=jax.ShapeDtypeStruct((B, T, N_E, D_E), jnp.float32),
        scratch_shapes=[
            pltpu.VMEM((NR, D_E), jnp.float32),
            pltpu.VMEM((TBLK, NR, GP), jnp.float32),
        ],
        compiler_params=pltpu.CompilerParams(
            dimension_semantics=("arbitrary",),
            vmem_limit_bytes=100 * 1024 * 1024,
        ),
    )(h_seq, mc, c, bih2, bhh2, whh_t, e0)

    entity_seq = stack.reshape(B, T, N_E * D_E)
    return entity_seq, stack


# speedup vs baseline: 1.1646x; 1.0092x over previous
import jax
import jax.numpy as jnp
from jax.experimental import pallas as pl
from jax.experimental.pallas import tpu as pltpu

B, T, D = 16, 2048, 1024
N_E, D_E = 8, 64
TBLK = 64
NT = T // TBLK
GP = 3 * 128
NR = B * N_E


def _entity_kernel(h_ref, mc_ref, c_ref, bih_ref, bhh_ref, whh_ref, e0_ref,
                   out_ref, state_ref, gxb_ref):
    j = pl.program_id(0)

    x2 = h_ref[...].reshape(B * TBLK, D).astype(jnp.bfloat16)
    mm = jnp.dot(x2, mc_ref[...], preferred_element_type=jnp.float32)
    lg = mm[:, GP:GP + N_E]
    m = jnp.max(lg, axis=-1, keepdims=True)
    p = jnp.exp(lg - m)
    w2 = p / jnp.sum(p, axis=-1, keepdims=True)
    w3 = w2.reshape(B, TBLK, N_E)

    for g in range(3):
        pg = mm[:, 128 * g:128 * g + D_E] + c_ref[:, 128 * g:128 * g + D_E]
        pre3g = pg.reshape(B, TBLK, D_E)
        gx4 = (w3[..., None] * pre3g[:, :, None, :]
               + bih_ref[:, 128 * g:128 * g + D_E])
        gxb_ref[:, :, 128 * g:128 * g + D_E] = (
            jnp.transpose(gx4, (1, 0, 2, 3)).reshape(TBLK, NR, D_E))

    @pl.when(j == 0)
    def _():
        state_ref[...] = jnp.concatenate([e0_ref[...]] * B, axis=0)

    def body(t, st):
        gx = gxb_ref[t]
        gh = jnp.dot(st.astype(jnp.bfloat16), whh_ref[...],
                     preferred_element_type=jnp.float32) + bhh_ref[...]
        s = gx + gh
        r = jax.nn.sigmoid(s[:, :D_E])
        z = jax.nn.sigmoid(s[:, 128:128 + D_E])
        n = jnp.tanh(gx[:, 256:256 + D_E] + r * gh[:, 256:256 + D_E])
        new = n + z * (st - n)
        out_ref[:, pl.ds(t, 1), :, :] = new.reshape(B, 1, N_E, D_E)
        return new

    st = jax.lax.fori_loop(0, TBLK, body, state_ref[...], unroll=16)
    state_ref[...] = st


def _pad_gates(a):
    z = jnp.zeros(a.shape[:-1] + (64,), a.dtype)
    return jnp.concatenate(
        [a[..., :64], z, a[..., 64:128], z, a[..., 128:192], z], axis=-1)


def kernel(h_seq, entity_keys, Wi, bi, W_ih, W_hh, b_ih, b_hh, e0):
    m_pre = _pad_gates((W_ih @ Wi).T)
    keys_t = entity_keys.T / jnp.sqrt(jnp.float32(D))
    mc = jnp.concatenate([m_pre, keys_t], axis=1).astype(jnp.bfloat16)
    c = _pad_gates((W_ih @ bi).reshape(1, 192))
    bih2 = _pad_gates(b_ih.reshape(1, 192))
    bhh2 = _pad_gates(b_hh.reshape(1, 192))
    whh_t = _pad_gates(W_hh.T).astype(jnp.bfloat16)

    stack = pl.pallas_call(
        _entity_kernel,
        grid=(NT,),
        in_specs=[
            pl.BlockSpec((B, TBLK, D), lambda j: (0, j, 0)),
            pl.BlockSpec((D, GP + N_E), lambda j: (0, 0)),
            pl.BlockSpec((1, GP), lambda j: (0, 0)),
            pl.BlockSpec((1, GP), lambda j: (0, 0)),
            pl.BlockSpec((1, GP), lambda j: (0, 0)),
            pl.BlockSpec((D_E, GP), lambda j: (0, 0)),
            pl.BlockSpec((N_E, D_E), lambda j: (0, 0)),
        ],
        out_specs=pl.BlockSpec((B, TBLK, N_E, D_E), lambda j: (0, j, 0, 0)),
        out_shape=jax.ShapeDtypeStruct((B, T, N_E, D_E), jnp.float32),
        scratch_shapes=[
            pltpu.VMEM((NR, D_E), jnp.float32),
            pltpu.VMEM((TBLK, NR, GP), jnp.float32),
        ],
        compiler_params=pltpu.CompilerParams(
            dimension_semantics=("arbitrary",),
            vmem_limit_bytes=100 * 1024 * 1024,
        ),
    )(h_seq, mc, c, bih2, bhh2, whh_t, e0)

    entity_seq = stack.reshape(B, T, N_E * D_E)
    return entity_seq, stack
